# baseline (speedup 1.0000x reference)
import functools

import jax
import jax.numpy as jnp
from jax import lax
from jax.experimental import pallas as pl
from jax.experimental.pallas import tpu as pltpu
from jax.experimental.pallas import tpu_sc as plsc

_NC = 2
_NS = 16
_NW = _NC * _NS
_NBUF = 2


@functools.lru_cache(maxsize=None)
def _make_gather(vocab, dim, batch, seq):
    assert batch % (_NW * _NBUF) == 0
    b_per_w = batch // _NW
    pieces = []
    off = 0
    while off < seq:
        n = min(128, seq - off)
        pieces.append((off, n))
        off += n
    mesh = plsc.VectorSubcoreMesh(core_axis_name="c", subcore_axis_name="s")

    @functools.partial(
        pl.kernel,
        out_type=jax.ShapeDtypeStruct((batch, seq, dim), jnp.float32),
        mesh=mesh,
        scratch_types=[
            pltpu.VMEM((b_per_w, seq), jnp.int32),
            pltpu.VMEM((_NBUF, seq, dim), jnp.float32),
            pltpu.SemaphoreType.DMA,
            pltpu.SemaphoreType.DMA,
        ],
        compiler_params=pltpu.CompilerParams(use_tc_tiling_on_sc=False),
    )
    def gather_kernel(idx_hbm, table_hbm, out_hbm, idx_v, rows_v, gsem, wsem):
        wid = lax.axis_index("s") * _NC + lax.axis_index("c")
        base_b = wid * b_per_w
        pltpu.sync_copy(idx_hbm.at[pl.ds(base_b, b_per_w)], idx_v)

        @pl.loop(0, b_per_w, step=_NBUF)
        def _task(t0):
            for h in range(_NBUF):
                t = t0 + h

                @pl.when(t0 > 0)
                def _():
                    pltpu.make_async_copy(rows_v.at[h], out_hbm.at[0], wsem).wait()

                descs = [
                    pltpu.async_copy(
                        table_hbm.at[idx_v.at[t].at[pl.ds(off, n)]],
                        rows_v.at[h].at[pl.ds(off, n)],
                        gsem,
                    )
                    for off, n in pieces
                ]
                for d in descs:
                    d.wait()
                pltpu.async_copy(rows_v.at[h], out_hbm.at[base_b + t], wsem)

        for h in range(_NBUF):
            pltpu.make_async_copy(rows_v.at[h], out_hbm.at[0], wsem).wait()

    return gather_kernel


def kernel(x, embeddings):
    batch, seq = x.shape
    vocab, dim = embeddings.shape
    return _make_gather(vocab, dim, batch, seq)(x.astype(jnp.int32), embeddings)
